# baseline (speedup 1.0000x reference)
import functools

import jax
import jax.numpy as jnp
from jax import lax
from jax.experimental import pallas as pl
from jax.experimental.pallas import tpu as pltpu
from jax.experimental.pallas import tpu_sc as plsc

EMB = 128
NC = 2
NW = 32
CR = 112
O1, O2, O3 = 119, 839, 1199
TROWS = 1203


def _sc_embed_sum(xq, tcat, n_pad, nch):
    mesh = plsc.VectorSubcoreMesh(core_axis_name="c", subcore_axis_name="s")
    rpt = nch * CR

    @functools.partial(
        pl.kernel,
        mesh=mesh,
        out_type=jax.ShapeDtypeStruct((n_pad, EMB), jnp.float32),
        scratch_types=[
            pltpu.VMEM((TROWS * EMB // 2,), jnp.int32),
            pltpu.VMEM((9 * CR,), jnp.int32),
            pltpu.VMEM((9 * CR,), jnp.int32),
            pltpu.VMEM((CR, EMB), jnp.float32),
            pltpu.VMEM((CR, EMB), jnp.float32),
            pltpu.SemaphoreType.DMA,
            pltpu.SemaphoreType.DMA,
            pltpu.SemaphoreType.DMA,
            pltpu.SemaphoreType.DMA,
        ],
    )
    def k(xq_hbm, t_hbm, out_hbm, tv, xv0, xv1, ob0, ob1, s0, s1, sx0, sx1):
        wid = lax.axis_index("s") * NC + lax.axis_index("c")
        pltpu.sync_copy(t_hbm, tv)
        xvs = (xv0, xv1)
        obs = (ob0, ob1)
        sems = (s0, s1)
        sxs = (sx0, sx1)
        xbase = wid * 9 * rpt

        def x_slice(ch):
            return xq_hbm.at[pl.ds(xbase + ch * 9 * CR, 9 * CR)]

        pltpu.async_copy(x_slice(0), xv0, sx0)
        pltpu.async_copy(x_slice(1), xv1, sx1)

        def compute_chunk(ob, xv):
            def jgroup(j, c2):
                b = j * 16
                hw = EMB // 2
                cv = [xv[pl.ds(cc * CR + b, 16)] for cc in range(9)]
                v0 = cv[0] * hw
                v1 = (O1 + (cv[1] * 12 + cv[2]) * 12 + cv[3]) * hw
                v2 = (O2 + (cv[4] * 6 + cv[5]) * 6 + cv[6]) * hw
                v3 = (O3 + cv[7] * 2 + cv[8]) * hw
                for l2 in range(8):
                    os_ = [
                        tuple(
                            pl.multiple_of(v[l2 * 2 + d], hw)
                            for v in (v0, v1, v2, v3)
                        )
                        for d in range(2)
                    ]
                    vals = [
                        [
                            [tv[pl.ds(os_[d][g] + c * 16, 16)] for g in range(4)]
                            for c in range(4)
                        ]
                        for d in range(2)
                    ]
                    for d in range(2):
                        for c in range(4):
                            q = vals[d][c]
                            lo = [
                                lax.bitcast_convert_type(w << 16, jnp.float32)
                                for w in q
                            ]
                            hi = [
                                lax.bitcast_convert_type(
                                    w & jnp.int32(-65536), jnp.float32
                                )
                                for w in q
                            ]
                            a0 = (lo[0] + lo[1]) + (lo[2] + lo[3])
                            a1 = (hi[0] + hi[1]) + (hi[2] + hi[3])
                            ob[b + l2 * 2 + d, pl.ds(c * 32, 16)] = a0
                            ob[b + l2 * 2 + d, pl.ds(c * 32 + 16, 16)] = a1
                return c2

            lax.fori_loop(0, CR // 16, jgroup, 0)

        def chunk2(i, carry):
            for p in range(2):
                ch = i * 2 + p
                pltpu.make_async_copy(x_slice(0), xvs[p], sxs[p]).wait()

                @pl.when(i > 0)
                def _wait_out():
                    pltpu.make_async_copy(
                        obs[p], out_hbm.at[pl.ds(wid * rpt, CR)], sems[p]
                    ).wait()

                compute_chunk(obs[p], xvs[p])
                pltpu.async_copy(
                    obs[p], out_hbm.at[pl.ds(wid * rpt + ch * CR, CR)], sems[p]
                )

                @pl.when(ch + 2 < nch)
                def _prefetch_x():
                    pltpu.async_copy(x_slice(ch + 2), xvs[p], sxs[p])
            return carry

        lax.fori_loop(0, nch // 2, chunk2, 0)
        for p in range(2):
            pltpu.make_async_copy(
                obs[p], out_hbm.at[pl.ds(wid * rpt, CR)], sems[p]
            ).wait()

    return k(xq, tcat)


def kernel(x, W0, W1, W2, W3, W4, W5, W6, W7, W8):
    n = x.shape[0]
    t123 = (
        W1[:, None, None, :] + W2[None, :, None, :] + W3[None, None, :, :]
    ).reshape(720, EMB)
    t456 = (
        W4[:, None, None, :] + W5[None, :, None, :] + W6[None, None, :, :]
    ).reshape(360, EMB)
    t78 = (W7[:, None, :] + W8[None, :, :]).reshape(4, EMB)
    tcat = jnp.concatenate([W0, t123, t456, t78], axis=0)
    order = []
    for blk in range(EMB // 32):
        for i in range(16):
            order.extend((blk * 32 + i, blk * 32 + 16 + i))
    tcat = tcat[:, jnp.array(order, dtype=jnp.int32)].astype(jnp.bfloat16)
    tcat = jax.lax.bitcast_convert_type(
        tcat.reshape(TROWS, EMB // 2, 2), jnp.int32
    ).reshape(-1)
    slab = NW * CR
    n_pad = ((n + slab - 1) // slab) * slab
    nch = n_pad // slab
    xp = jnp.pad(x, ((0, n_pad - n), (0, 0)))
    xq = xp.reshape(NW, nch, CR, 9).transpose(0, 1, 3, 2).reshape(-1)
    out = _sc_embed_sum(xq, tcat, n_pad, nch)
    return out[:n]


# speedup vs baseline: 1.0948x; 1.0948x over previous
import functools

import jax
import jax.numpy as jnp
from jax import lax
from jax.experimental import pallas as pl
from jax.experimental.pallas import tpu as pltpu
from jax.experimental.pallas import tpu_sc as plsc

EMB = 128
NC = 2
NW = 32
CR = 112
O1, O2, O3, O4 = 119, 179, 299, 335
TROWS = 339


def _sc_embed_sum(xq, tcat, n_pad, nch):
    mesh = plsc.VectorSubcoreMesh(core_axis_name="c", subcore_axis_name="s")
    rpt = nch * CR

    @functools.partial(
        pl.kernel,
        mesh=mesh,
        out_type=jax.ShapeDtypeStruct((n_pad, EMB), jnp.float32),
        scratch_types=[
            pltpu.VMEM((TROWS * EMB // 2,), jnp.int32),
            pltpu.VMEM((9 * rpt,), jnp.int32),
            pltpu.VMEM((CR, EMB), jnp.float32),
            pltpu.VMEM((CR, EMB), jnp.float32),
            pltpu.SemaphoreType.DMA,
            pltpu.SemaphoreType.DMA,
        ],
    )
    def k(xq_hbm, t_hbm, out_hbm, tv, xv, ob0, ob1, s0, s1):
        wid = lax.axis_index("s") * NC + lax.axis_index("c")
        pltpu.sync_copy(t_hbm, tv)
        pltpu.sync_copy(xq_hbm.at[pl.ds(wid * 9 * rpt, 9 * rpt)], xv)
        obs = (ob0, ob1)
        sems = (s0, s1)

        def compute_chunk(ch, ob):
            def jgroup(j, c2):
                b = ch * CR + j * 16
                hw = EMB // 2
                cv = [xv[pl.ds(cc * rpt + b, 16)] for cc in range(9)]
                v0 = cv[0] * hw
                v1 = (O1 + cv[1] * 12 + cv[2]) * hw
                v2 = (O2 + cv[3] * 10 + cv[4]) * hw
                v3 = (O3 + cv[5] * 6 + cv[6]) * hw
                v4 = (O4 + cv[7] * 2 + cv[8]) * hw
                for l2 in range(8):
                    os = [
                        tuple(
                            pl.multiple_of(v[l2 * 2 + d], hw)
                            for v in (v0, v1, v2, v3, v4)
                        )
                        for d in range(2)
                    ]
                    vals = [
                        [
                            [tv[pl.ds(os[d][g] + c * 16, 16)] for g in range(5)]
                            for c in range(4)
                        ]
                        for d in range(2)
                    ]
                    for d in range(2):
                        for c in range(4):
                            q = vals[d][c]
                            lo = [
                                lax.bitcast_convert_type(w << 16, jnp.float32)
                                for w in q
                            ]
                            hi = [
                                lax.bitcast_convert_type(w, jnp.float32)
                                for w in q
                            ]
                            a0 = ((lo[0] + lo[1]) + (lo[2] + lo[3])) + lo[4]
                            a1 = ((hi[0] + hi[1]) + (hi[2] + hi[3])) + hi[4]
                            ob[j * 16 + l2 * 2 + d, pl.ds(c * 32, 16)] = a0
                            ob[j * 16 + l2 * 2 + d, pl.ds(c * 32 + 16, 16)] = a1
                return c2

            lax.fori_loop(0, CR // 16, jgroup, 0)

        def chunk2(i, carry):
            for p in range(2):
                ch = i * 2 + p

                @pl.when(i > 0)
                def _wait():
                    pltpu.make_async_copy(
                        obs[p], out_hbm.at[pl.ds(wid * rpt, CR)], sems[p]
                    ).wait()

                compute_chunk(ch, obs[p])
                pltpu.async_copy(
                    obs[p], out_hbm.at[pl.ds(wid * rpt + ch * CR, CR)], sems[p]
                )
            return carry

        lax.fori_loop(0, nch // 2, chunk2, 0)
        for p in range(2):
            pltpu.make_async_copy(
                obs[p], out_hbm.at[pl.ds(wid * rpt, CR)], sems[p]
            ).wait()

    return k(xq, tcat)


def kernel(x, W0, W1, W2, W3, W4, W5, W6, W7, W8):
    n = x.shape[0]
    t12 = (W1[:, None, :] + W2[None, :, :]).reshape(60, EMB)
    t34 = (W3[:, None, :] + W4[None, :, :]).reshape(120, EMB)
    t56 = (W5[:, None, :] + W6[None, :, :]).reshape(36, EMB)
    t78 = (W7[:, None, :] + W8[None, :, :]).reshape(4, EMB)
    tcat = jnp.concatenate([W0, t12, t34, t56, t78], axis=0)
    order = []
    for blk in range(EMB // 32):
        for i in range(16):
            order.extend((blk * 32 + i, blk * 32 + 16 + i))
    tcat = tcat[:, jnp.array(order, dtype=jnp.int32)].astype(jnp.bfloat16)
    tcat = jax.lax.bitcast_convert_type(
        tcat.reshape(TROWS, EMB // 2, 2), jnp.int32
    ).reshape(-1)
    slab = NW * CR
    n_pad = ((n + slab - 1) // slab) * slab
    nch = n_pad // slab
    xp = jnp.pad(x, ((0, n_pad - n), (0, 0)))
    xq = xp.reshape(NW, nch * CR, 9).transpose(0, 2, 1).reshape(-1)
    out = _sc_embed_sum(xq, tcat, n_pad, nch)
    return out[:n]
